# speedup vs baseline: 1.0862x; 1.0862x over previous
import functools

import jax
import jax.numpy as jnp
from jax import lax
from jax.experimental import pallas as pl
from jax.experimental.pallas import tpu as pltpu
from jax.experimental.pallas import tpu_sc as plsc

NUM_IDX = 16384
ROW = 128
NROWS = NUM_IDX // ROW
_info = plsc.get_sparse_core_info()
NC, NS = _info.num_cores, _info.num_subcores
NW = NC * NS
ROWS_PER_W = NROWS // NW

_mesh = plsc.VectorSubcoreMesh(core_axis_name="c", subcore_axis_name="s")


@functools.partial(
    pl.kernel,
    mesh=_mesh,
    out_type=jax.ShapeDtypeStruct((NROWS, ROW), jnp.float32),
    scratch_types=[
        pltpu.VMEM((ROWS_PER_W, ROW), jnp.int32),
        pltpu.VMEM((ROWS_PER_W, ROW), jnp.float32),
        pltpu.SemaphoreType.DMA,
    ],
)
def _sc_gather(probs_hbm, idx_hbm, out_hbm, idx_v, vals_v, sem):
    wid = lax.axis_index("s") * NC + lax.axis_index("c")
    base = wid * ROWS_PER_W
    pltpu.sync_copy(idx_hbm.at[pl.ds(base, ROWS_PER_W)], idx_v)
    copies = [
        pltpu.async_copy(probs_hbm.at[idx_v.at[j]], vals_v.at[j], sem)
        for j in range(ROWS_PER_W)
    ]
    for c in copies:
        c.wait()
    pltpu.sync_copy(vals_v, out_hbm.at[pl.ds(base, ROWS_PER_W)])


def _tc_finalize(p_ref, g_ref, o_ref):
    s = jnp.sum(p_ref[...])
    o_ref[...] = jnp.log(g_ref[...] / s)


def kernel(probs, indices):
    idx2d = indices.astype(jnp.int32).reshape(NROWS, ROW)
    gathered = _sc_gather(probs, idx2d)
    out = pl.pallas_call(
        _tc_finalize,
        out_shape=jax.ShapeDtypeStruct((NROWS, ROW), jnp.float32),
    )(probs.reshape(1000, 1000), gathered)
    return out.reshape(NUM_IDX)
